# baseline (speedup 1.0000x reference)
import functools

import jax
import jax.numpy as jnp
from jax import lax
from jax.experimental import pallas as pl
from jax.experimental.pallas import tpu as pltpu
from jax.experimental.pallas import tpu_sc as plsc

HIDDEN = 768
PAD = 1
EPS = 1e-5
MAX_POS = 2050

NC, NS, L = 2, 16, 16
NW = NC * NS
B = 4 * 2048
ROWS_PER_W = B // NW
CHUNK = 16
NCHUNK = ROWS_PER_W // CHUNK
NG = HIDDEN // L


def _gs(g):
    return pl.ds(g * L, L)


def _vrsqrt(v):
    i = plsc.bitcast(v, jnp.int32)
    i = jnp.int32(0x5F3759DF) - (i >> 1)
    y = plsc.bitcast(i, jnp.float32)
    for _ in range(3):
        y = y * (1.5 - 0.5 * v * y * y)
    return y


def _fold_body(pos_ref, tt_ref, o_ref):
    o_ref[...] = pos_ref[...] + tt_ref[0:1, :]


@jax.jit
def _fold_tt(ptab, tttab):
    return pl.pallas_call(
        _fold_body,
        out_shape=jax.ShapeDtypeStruct((MAX_POS, HIDDEN), jnp.float32),
    )(ptab, tttab)


def _sc_body(ids_hbm, wtab_hbm, ptt_hbm, gamma_hbm, beta_hbm,
             out_hbm,
             ids_v, wida, pida, gam_v, bet_v, pref_v,
             wbuf0, wbuf1, pbuf0, pbuf1, obuf0, obuf1,
             wsem0, wsem1, psem0, psem1, osem0, osem1):
    c = lax.axis_index("c")
    s = lax.axis_index("s")
    wid = c * NS + s
    chk = wid % 8
    base_tok = wid * ROWS_PER_W
    seq_tok = base_tok - chk * ROWS_PER_W

    pltpu.sync_copy(ids_hbm.at[pl.ds(base_tok, ROWS_PER_W)], ids_v)
    pltpu.sync_copy(gamma_hbm, gam_v)
    pltpu.sync_copy(beta_hbm, bet_v)

    def jbody(j, acc):
        pltpu.sync_copy(ids_hbm.at[pl.ds(seq_tok + j * ROWS_PER_W,
                                         ROWS_PER_W)], pref_v)

        def cbody(i, a):
            v = pref_v[pl.ds(i * L, L)]
            return a + jnp.where(v != PAD, 1, 0).astype(jnp.int32)

        return lax.fori_loop(0, ROWS_PER_W // L, cbody, acc)

    off = lax.fori_loop(0, chk, jbody, jnp.zeros((L,), jnp.int32))
    off = jnp.full((L,), jnp.sum(off), jnp.int32)

    def pbody(i, run):
        v = ids_v[pl.ds(i * L, L)]
        m = jnp.where(v != PAD, 1, 0).astype(jnp.int32)
        cs = plsc.cumsum(m)
        wida[i] = v
        pida[i] = (cs + run) * m + PAD
        return run + jnp.sum(m)

    lax.fori_loop(0, NCHUNK, pbody, off)

    wbufs = (wbuf0, wbuf1)
    pbufs = (pbuf0, pbuf1)
    obufs = (obuf0, obuf1)
    wsems = (wsem0, wsem1)
    psems = (psem0, psem1)
    osems = (osem0, osem1)
    inv_h = jnp.float32(1.0 / HIDDEN)
    z = jnp.zeros((L,), jnp.float32)

    def chunk_step(k, b):
        wb, pb, ob = wbufs[b], pbufs[b], obufs[b]
        pltpu.make_async_copy(ob, out_hbm.at[pl.ds(base_tok, CHUNK)],
                              osems[b]).wait()
        pltpu.make_async_copy(wtab_hbm.at[wida.at[k]], wb, wsems[b]).wait()
        pltpu.make_async_copy(ptt_hbm.at[pida.at[k]], pb, psems[b]).wait()

        @plsc.parallel_loop(0, CHUNK, 2)
        def rbody(r):
            ra = r
            rb = r + 1
            sa0 = sa1 = qa0 = qa1 = z
            sb0 = sb1 = qb0 = qb1 = z
            for g in range(NG):
                xa = wb[ra, _gs(g)] + pb[ra, _gs(g)]
                xb = wb[rb, _gs(g)] + pb[rb, _gs(g)]
                pb[ra, _gs(g)] = xa
                pb[rb, _gs(g)] = xb
                if g % 2 == 0:
                    sa0 = sa0 + xa
                    qa0 = qa0 + xa * xa
                    sb0 = sb0 + xb
                    qb0 = qb0 + xb * xb
                else:
                    sa1 = sa1 + xa
                    qa1 = qa1 + xa * xa
                    sb1 = sb1 + xb
                    qb1 = qb1 + xb * xb
            meana = jnp.full((L,), jnp.sum(sa0 + sa1) * inv_h, jnp.float32)
            meanb = jnp.full((L,), jnp.sum(sb0 + sb1) * inv_h, jnp.float32)
            ex2a = jnp.full((L,), jnp.sum(qa0 + qa1) * inv_h, jnp.float32)
            ex2b = jnp.full((L,), jnp.sum(qb0 + qb1) * inv_h, jnp.float32)
            rinva = _vrsqrt(ex2a - meana * meana + EPS)
            rinvb = _vrsqrt(ex2b - meanb * meanb + EPS)
            mba = meana * rinva
            mbb = meanb * rinvb
            for g in range(NG):
                gv = gam_v[_gs(g)]
                bv = bet_v[_gs(g)]
                ta = pb[ra, _gs(g)] * rinva - mba
                tb = pb[rb, _gs(g)] * rinvb - mbb
                ob[ra, _gs(g)] = ta * gv + bv
                ob[rb, _gs(g)] = tb * gv + bv

        pltpu.async_copy(ob, out_hbm.at[pl.ds(base_tok + k * CHUNK, CHUNK)],
                         osems[b])
        k2 = (k + 2) % NCHUNK
        pltpu.async_copy(wtab_hbm.at[wida.at[k2]], wb, wsems[b])
        pltpu.async_copy(ptt_hbm.at[pida.at[k2]], pb, psems[b])

    pltpu.async_copy(wtab_hbm.at[wida.at[0]], wbuf0, wsem0)
    pltpu.async_copy(ptt_hbm.at[pida.at[0]], pbuf0, psem0)
    pltpu.async_copy(wtab_hbm.at[wida.at[1]], wbuf1, wsem1)
    pltpu.async_copy(ptt_hbm.at[pida.at[1]], pbuf1, psem1)
    pltpu.async_copy(obuf0, out_hbm.at[pl.ds(base_tok, CHUNK)], osem0)
    pltpu.async_copy(obuf1, out_hbm.at[pl.ds(base_tok, CHUNK)], osem1)

    def loop_body(i, _):
        k = 2 * i
        chunk_step(k, 0)
        chunk_step(k + 1, 1)
        return 0

    lax.fori_loop(0, NCHUNK // 2, loop_body, 0)

    pltpu.make_async_copy(wtab_hbm.at[wida.at[0]], wbuf0, wsem0).wait()
    pltpu.make_async_copy(ptt_hbm.at[pida.at[0]], pbuf0, psem0).wait()
    pltpu.make_async_copy(wtab_hbm.at[wida.at[1]], wbuf1, wsem1).wait()
    pltpu.make_async_copy(ptt_hbm.at[pida.at[1]], pbuf1, psem1).wait()
    pltpu.make_async_copy(obuf0, out_hbm.at[pl.ds(base_tok, CHUNK)],
                          osem0).wait()
    pltpu.make_async_copy(obuf1, out_hbm.at[pl.ds(base_tok, CHUNK)],
                          osem1).wait()


@jax.jit
def _run(ids, wtab, ptt, gamma, beta):
    mesh = plsc.VectorSubcoreMesh(
        core_axis_name="c", subcore_axis_name="s",
        num_cores=NC, num_subcores=NS)
    f = pl.kernel(
        _sc_body,
        out_type=jax.ShapeDtypeStruct((B, HIDDEN), jnp.float32),
        mesh=mesh,
        compiler_params=pltpu.CompilerParams(needs_layout_passes=False),
        scratch_types=[
            pltpu.VMEM((ROWS_PER_W,), jnp.int32),
            pltpu.VMEM((NCHUNK, CHUNK), jnp.int32),
            pltpu.VMEM((NCHUNK, CHUNK), jnp.int32),
            pltpu.VMEM((HIDDEN,), jnp.float32),
            pltpu.VMEM((HIDDEN,), jnp.float32),
            pltpu.VMEM((ROWS_PER_W,), jnp.int32),
            pltpu.VMEM((CHUNK, HIDDEN), jnp.float32),
            pltpu.VMEM((CHUNK, HIDDEN), jnp.float32),
            pltpu.VMEM((CHUNK, HIDDEN), jnp.float32),
            pltpu.VMEM((CHUNK, HIDDEN), jnp.float32),
            pltpu.VMEM((CHUNK, HIDDEN), jnp.float32),
            pltpu.VMEM((CHUNK, HIDDEN), jnp.float32),
            pltpu.SemaphoreType.DMA,
            pltpu.SemaphoreType.DMA,
            pltpu.SemaphoreType.DMA,
            pltpu.SemaphoreType.DMA,
            pltpu.SemaphoreType.DMA,
            pltpu.SemaphoreType.DMA,
        ],
    )
    return f(ids, wtab, ptt, gamma, beta)


def kernel(input_ids, word_embeddings, position_embeddings,
           token_type_embeddings, ln_gamma, ln_beta):
    ids = input_ids.reshape(-1).astype(jnp.int32)
    ptt = _fold_tt(position_embeddings, token_type_embeddings)
    out = _run(ids, word_embeddings, ptt, ln_gamma, ln_beta)
    return out.reshape(input_ids.shape + (HIDDEN,))


# speedup vs baseline: 1.5378x; 1.5378x over previous
import functools

import jax
import jax.numpy as jnp
from jax import lax
from jax.experimental import pallas as pl
from jax.experimental.pallas import tpu as pltpu
from jax.experimental.pallas import tpu_sc as plsc

HIDDEN = 768
PAD = 1
EPS = 1e-5
MAX_POS = 2050

NC, NS, L = 2, 16, 16
NW = NC * NS
B = 4 * 2048
ROWS_PER_W = B // NW
CHUNK = 16
NCHUNK = ROWS_PER_W // CHUNK
NG = HIDDEN // L


def _gs(g):
    return pl.ds(g * L, L)


def _vrsqrt(v):
    i = plsc.bitcast(v, jnp.int32)
    i = jnp.int32(0x5F3759DF) - (i >> 1)
    y = plsc.bitcast(i, jnp.float32)
    for _ in range(3):
        y = y * (1.5 - 0.5 * v * y * y)
    return y


def _fold_body(pos_ref, tt_ref, o_ref):
    o_ref[...] = pos_ref[...] + tt_ref[0:1, :]


@jax.jit
def _fold_tt(ptab, tttab):
    return pl.pallas_call(
        _fold_body,
        out_shape=jax.ShapeDtypeStruct((MAX_POS, HIDDEN), jnp.float32),
    )(ptab, tttab)


def _sc_body(ids_hbm, wtab_hbm, ptt_hbm, gamma_hbm, beta_hbm,
             out_hbm,
             ids_v, wida, pida, gam_v, bet_v, pref_v,
             wbuf0, wbuf1, pbuf0, pbuf1, obuf0, obuf1,
             wsem0, wsem1, psem0, psem1, osem0, osem1):
    c = lax.axis_index("c")
    s = lax.axis_index("s")
    wid = c * NS + s
    chk = wid % 8
    base_tok = wid * ROWS_PER_W
    seq_tok = base_tok - chk * ROWS_PER_W

    pltpu.sync_copy(ids_hbm.at[pl.ds(base_tok, ROWS_PER_W)], ids_v)
    pltpu.sync_copy(gamma_hbm, gam_v)
    pltpu.sync_copy(beta_hbm, bet_v)

    def jbody(j, acc):
        pltpu.sync_copy(ids_hbm.at[pl.ds(seq_tok + j * ROWS_PER_W,
                                         ROWS_PER_W)], pref_v)

        def cbody(i, a):
            v = pref_v[pl.ds(i * L, L)]
            return a + jnp.where(v != PAD, 1, 0).astype(jnp.int32)

        return lax.fori_loop(0, ROWS_PER_W // L, cbody, acc)

    off = lax.fori_loop(0, chk, jbody, jnp.zeros((L,), jnp.int32))
    off = jnp.full((L,), jnp.sum(off), jnp.int32)

    def pbody(i, run):
        v = ids_v[pl.ds(i * L, L)]
        m = jnp.where(v != PAD, 1, 0).astype(jnp.int32)
        cs = plsc.cumsum(m)
        wida[i] = v
        pida[i] = (cs + run) * m + PAD
        return run + jnp.sum(m)

    lax.fori_loop(0, NCHUNK, pbody, off)

    wbufs = (wbuf0, wbuf1)
    pbufs = (pbuf0, pbuf1)
    obufs = (obuf0, obuf1)
    wsems = (wsem0, wsem1)
    psems = (psem0, psem1)
    osems = (osem0, osem1)
    inv_h = jnp.float32(1.0 / HIDDEN)
    z = jnp.zeros((L,), jnp.float32)

    def chunk_step(k, b):
        wb, pb, ob = wbufs[b], pbufs[b], obufs[b]
        pltpu.make_async_copy(ob, out_hbm.at[pl.ds(base_tok, CHUNK)],
                              osems[b]).wait()
        pltpu.make_async_copy(wtab_hbm.at[wida.at[k]], wb, wsems[b]).wait()
        pltpu.make_async_copy(ptt_hbm.at[pida.at[k]], pb, psems[b]).wait()

        BW = 4

        def rbody(r, _):
            xs = []
            s0 = s1 = s2 = s3 = z
            q0 = q1 = q2 = q3 = z
            for gb in range(0, NG, BW):
                ws = [wb[r, _gs(g)] for g in range(gb, gb + BW)]
                ps = [pb[r, _gs(g)] for g in range(gb, gb + BW)]
                xb4 = [w + p for w, p in zip(ws, ps)]
                xs.extend(xb4)
                sq = [x * x for x in xb4]
                s0 = s0 + xb4[0]
                s1 = s1 + xb4[1]
                s2 = s2 + xb4[2]
                s3 = s3 + xb4[3]
                q0 = q0 + sq[0]
                q1 = q1 + sq[1]
                q2 = q2 + sq[2]
                q3 = q3 + sq[3]
            mean = jnp.full((L,), jnp.sum((s0 + s1) + (s2 + s3)) * inv_h,
                            jnp.float32)
            ex2 = jnp.full((L,), jnp.sum((q0 + q1) + (q2 + q3)) * inv_h,
                           jnp.float32)
            rinv = _vrsqrt(ex2 - mean * mean + EPS)
            mb = mean * rinv
            for gb in range(0, NG, BW):
                gvs = [gam_v[_gs(g)] for g in range(gb, gb + BW)]
                bvs = [bet_v[_gs(g)] for g in range(gb, gb + BW)]
                ts = [xs[g] * rinv - mb for g in range(gb, gb + BW)]
                ys = [t * gv + bv for t, gv, bv in zip(ts, gvs, bvs)]
                for i, g in enumerate(range(gb, gb + BW)):
                    ob[r, _gs(g)] = ys[i]
            return 0

        lax.fori_loop(0, CHUNK, rbody, 0)

        pltpu.async_copy(ob, out_hbm.at[pl.ds(base_tok + k * CHUNK, CHUNK)],
                         osems[b])
        k2 = (k + 2) % NCHUNK
        pltpu.async_copy(wtab_hbm.at[wida.at[k2]], wb, wsems[b])
        pltpu.async_copy(ptt_hbm.at[pida.at[k2]], pb, psems[b])

    pltpu.async_copy(wtab_hbm.at[wida.at[0]], wbuf0, wsem0)
    pltpu.async_copy(ptt_hbm.at[pida.at[0]], pbuf0, psem0)
    pltpu.async_copy(wtab_hbm.at[wida.at[1]], wbuf1, wsem1)
    pltpu.async_copy(ptt_hbm.at[pida.at[1]], pbuf1, psem1)
    pltpu.async_copy(obuf0, out_hbm.at[pl.ds(base_tok, CHUNK)], osem0)
    pltpu.async_copy(obuf1, out_hbm.at[pl.ds(base_tok, CHUNK)], osem1)

    def loop_body(i, _):
        k = 2 * i
        chunk_step(k, 0)
        chunk_step(k + 1, 1)
        return 0

    lax.fori_loop(0, NCHUNK // 2, loop_body, 0)

    pltpu.make_async_copy(wtab_hbm.at[wida.at[0]], wbuf0, wsem0).wait()
    pltpu.make_async_copy(ptt_hbm.at[pida.at[0]], pbuf0, psem0).wait()
    pltpu.make_async_copy(wtab_hbm.at[wida.at[1]], wbuf1, wsem1).wait()
    pltpu.make_async_copy(ptt_hbm.at[pida.at[1]], pbuf1, psem1).wait()
    pltpu.make_async_copy(obuf0, out_hbm.at[pl.ds(base_tok, CHUNK)],
                          osem0).wait()
    pltpu.make_async_copy(obuf1, out_hbm.at[pl.ds(base_tok, CHUNK)],
                          osem1).wait()


@jax.jit
def _run(ids, wtab, ptt, gamma, beta):
    mesh = plsc.VectorSubcoreMesh(
        core_axis_name="c", subcore_axis_name="s",
        num_cores=NC, num_subcores=NS)
    f = pl.kernel(
        _sc_body,
        out_type=jax.ShapeDtypeStruct((B, HIDDEN), jnp.float32),
        mesh=mesh,
        compiler_params=pltpu.CompilerParams(needs_layout_passes=False),
        scratch_types=[
            pltpu.VMEM((ROWS_PER_W,), jnp.int32),
            pltpu.VMEM((NCHUNK, CHUNK), jnp.int32),
            pltpu.VMEM((NCHUNK, CHUNK), jnp.int32),
            pltpu.VMEM((HIDDEN,), jnp.float32),
            pltpu.VMEM((HIDDEN,), jnp.float32),
            pltpu.VMEM((ROWS_PER_W,), jnp.int32),
            pltpu.VMEM((CHUNK, HIDDEN), jnp.float32),
            pltpu.VMEM((CHUNK, HIDDEN), jnp.float32),
            pltpu.VMEM((CHUNK, HIDDEN), jnp.float32),
            pltpu.VMEM((CHUNK, HIDDEN), jnp.float32),
            pltpu.VMEM((CHUNK, HIDDEN), jnp.float32),
            pltpu.VMEM((CHUNK, HIDDEN), jnp.float32),
            pltpu.SemaphoreType.DMA,
            pltpu.SemaphoreType.DMA,
            pltpu.SemaphoreType.DMA,
            pltpu.SemaphoreType.DMA,
            pltpu.SemaphoreType.DMA,
            pltpu.SemaphoreType.DMA,
        ],
    )
    return f(ids, wtab, ptt, gamma, beta)


def kernel(input_ids, word_embeddings, position_embeddings,
           token_type_embeddings, ln_gamma, ln_beta):
    ids = input_ids.reshape(-1).astype(jnp.int32)
    ptt = _fold_tt(position_embeddings, token_type_embeddings)
    out = _run(ids, word_embeddings, ptt, ln_gamma, ln_beta)
    return out.reshape(input_ids.shape + (HIDDEN,))


# speedup vs baseline: 1.6568x; 1.0774x over previous
import functools

import jax
import jax.numpy as jnp
from jax import lax
from jax.experimental import pallas as pl
from jax.experimental.pallas import tpu as pltpu
from jax.experimental.pallas import tpu_sc as plsc

HIDDEN = 768
PAD = 1
EPS = 1e-5
MAX_POS = 2050

NC, NS, L = 2, 16, 16
NW = NC * NS
B = 4 * 2048
ROWS_PER_W = B // NW
CHUNK = 16
NCHUNK = ROWS_PER_W // CHUNK
NG = HIDDEN // L


def _gs(g):
    return pl.ds(g * L, L)


def _vrsqrt(v):
    i = plsc.bitcast(v, jnp.int32)
    i = jnp.int32(0x5F3759DF) - (i >> 1)
    y = plsc.bitcast(i, jnp.float32)
    for _ in range(3):
        y = y * (1.5 - 0.5 * v * y * y)
    return y


def _fold_body(pos_ref, tt_ref, o_ref):
    o_ref[...] = pos_ref[...] + tt_ref[0:1, :]


@jax.jit
def _fold_tt(ptab, tttab):
    return pl.pallas_call(
        _fold_body,
        out_shape=jax.ShapeDtypeStruct((MAX_POS, HIDDEN), jnp.float32),
    )(ptab, tttab)


def _sc_body(ids_hbm, wtab_hbm, ptt_hbm, gamma_hbm, beta_hbm,
             out_hbm,
             ids_v, wida, pida, gam_v, bet_v, pref_v,
             wbuf0, wbuf1, pbuf0, pbuf1, obuf0, obuf1,
             wsem0, wsem1, psem0, psem1, osem0, osem1):
    c = lax.axis_index("c")
    s = lax.axis_index("s")
    wid = c * NS + s
    chk = wid % 8
    base_tok = wid * ROWS_PER_W
    seq_tok = base_tok - chk * ROWS_PER_W

    pltpu.sync_copy(ids_hbm.at[pl.ds(base_tok, ROWS_PER_W)], ids_v)
    pltpu.sync_copy(gamma_hbm, gam_v)
    pltpu.sync_copy(beta_hbm, bet_v)

    def jbody(j, acc):
        pltpu.sync_copy(ids_hbm.at[pl.ds(seq_tok + j * ROWS_PER_W,
                                         ROWS_PER_W)], pref_v)

        def cbody(i, a):
            v = pref_v[pl.ds(i * L, L)]
            return a + jnp.where(v != PAD, 1, 0).astype(jnp.int32)

        return lax.fori_loop(0, ROWS_PER_W // L, cbody, acc)

    off = lax.fori_loop(0, chk, jbody, jnp.zeros((L,), jnp.int32))
    off = jnp.full((L,), jnp.sum(off), jnp.int32)

    def pbody(i, run):
        v = ids_v[pl.ds(i * L, L)]
        m = jnp.where(v != PAD, 1, 0).astype(jnp.int32)
        cs = plsc.cumsum(m)
        wida[i] = v
        pida[i] = (cs + run) * m + PAD
        return run + jnp.sum(m)

    lax.fori_loop(0, NCHUNK, pbody, off)

    wbufs = (wbuf0, wbuf1)
    pbufs = (pbuf0, pbuf1)
    obufs = (obuf0, obuf1)
    wsems = (wsem0, wsem1)
    psems = (psem0, psem1)
    osems = (osem0, osem1)
    inv_h = jnp.float32(1.0 / HIDDEN)
    z = jnp.zeros((L,), jnp.float32)

    def chunk_step(k, b):
        wb, pb, ob = wbufs[b], pbufs[b], obufs[b]
        pltpu.make_async_copy(ob, out_hbm.at[pl.ds(base_tok, CHUNK)],
                              osems[b]).wait()
        pltpu.make_async_copy(wtab_hbm.at[wida.at[k]], wb, wsems[b]).wait()
        pltpu.make_async_copy(ptt_hbm.at[pida.at[k]], pb, psems[b]).wait()

        BW = 8

        def rbody(r, _):
            xs = []
            s0 = s1 = s2 = s3 = z
            q0 = q1 = q2 = q3 = z
            for gb in range(0, NG, BW):
                ws = [wb[r, _gs(g)] for g in range(gb, gb + BW)]
                ps = [pb[r, _gs(g)] for g in range(gb, gb + BW)]
                xb4 = [w + p for w, p in zip(ws, ps)]
                xs.extend(xb4)
                sq = [x * x for x in xb4]
                s0 = s0 + (xb4[0] + xb4[4])
                s1 = s1 + (xb4[1] + xb4[5])
                s2 = s2 + (xb4[2] + xb4[6])
                s3 = s3 + (xb4[3] + xb4[7])
                q0 = q0 + (sq[0] + sq[4])
                q1 = q1 + (sq[1] + sq[5])
                q2 = q2 + (sq[2] + sq[6])
                q3 = q3 + (sq[3] + sq[7])
            mean = jnp.full((L,), jnp.sum((s0 + s1) + (s2 + s3)) * inv_h,
                            jnp.float32)
            ex2 = jnp.full((L,), jnp.sum((q0 + q1) + (q2 + q3)) * inv_h,
                           jnp.float32)
            rinv = _vrsqrt(ex2 - mean * mean + EPS)
            mb = mean * rinv
            for gb in range(0, NG, BW):
                gvs = [gam_v[_gs(g)] for g in range(gb, gb + BW)]
                bvs = [bet_v[_gs(g)] for g in range(gb, gb + BW)]
                ts = [xs[g] * rinv - mb for g in range(gb, gb + BW)]
                ys = [t * gv + bv for t, gv, bv in zip(ts, gvs, bvs)]
                for i, g in enumerate(range(gb, gb + BW)):
                    ob[r, _gs(g)] = ys[i]
            return 0

        lax.fori_loop(0, CHUNK, rbody, 0)

        pltpu.async_copy(ob, out_hbm.at[pl.ds(base_tok + k * CHUNK, CHUNK)],
                         osems[b])
        k2 = (k + 2) % NCHUNK
        pltpu.async_copy(wtab_hbm.at[wida.at[k2]], wb, wsems[b])
        pltpu.async_copy(ptt_hbm.at[pida.at[k2]], pb, psems[b])

    pltpu.async_copy(wtab_hbm.at[wida.at[0]], wbuf0, wsem0)
    pltpu.async_copy(ptt_hbm.at[pida.at[0]], pbuf0, psem0)
    pltpu.async_copy(wtab_hbm.at[wida.at[1]], wbuf1, wsem1)
    pltpu.async_copy(ptt_hbm.at[pida.at[1]], pbuf1, psem1)
    pltpu.async_copy(obuf0, out_hbm.at[pl.ds(base_tok, CHUNK)], osem0)
    pltpu.async_copy(obuf1, out_hbm.at[pl.ds(base_tok, CHUNK)], osem1)

    def loop_body(i, _):
        k = 2 * i
        chunk_step(k, 0)
        chunk_step(k + 1, 1)
        return 0

    lax.fori_loop(0, NCHUNK // 2, loop_body, 0)

    pltpu.make_async_copy(wtab_hbm.at[wida.at[0]], wbuf0, wsem0).wait()
    pltpu.make_async_copy(ptt_hbm.at[pida.at[0]], pbuf0, psem0).wait()
    pltpu.make_async_copy(wtab_hbm.at[wida.at[1]], wbuf1, wsem1).wait()
    pltpu.make_async_copy(ptt_hbm.at[pida.at[1]], pbuf1, psem1).wait()
    pltpu.make_async_copy(obuf0, out_hbm.at[pl.ds(base_tok, CHUNK)],
                          osem0).wait()
    pltpu.make_async_copy(obuf1, out_hbm.at[pl.ds(base_tok, CHUNK)],
                          osem1).wait()


@jax.jit
def _run(ids, wtab, ptt, gamma, beta):
    mesh = plsc.VectorSubcoreMesh(
        core_axis_name="c", subcore_axis_name="s",
        num_cores=NC, num_subcores=NS)
    f = pl.kernel(
        _sc_body,
        out_type=jax.ShapeDtypeStruct((B, HIDDEN), jnp.float32),
        mesh=mesh,
        compiler_params=pltpu.CompilerParams(needs_layout_passes=False),
        scratch_types=[
            pltpu.VMEM((ROWS_PER_W,), jnp.int32),
            pltpu.VMEM((NCHUNK, CHUNK), jnp.int32),
            pltpu.VMEM((NCHUNK, CHUNK), jnp.int32),
            pltpu.VMEM((HIDDEN,), jnp.float32),
            pltpu.VMEM((HIDDEN,), jnp.float32),
            pltpu.VMEM((ROWS_PER_W,), jnp.int32),
            pltpu.VMEM((CHUNK, HIDDEN), jnp.float32),
            pltpu.VMEM((CHUNK, HIDDEN), jnp.float32),
            pltpu.VMEM((CHUNK, HIDDEN), jnp.float32),
            pltpu.VMEM((CHUNK, HIDDEN), jnp.float32),
            pltpu.VMEM((CHUNK, HIDDEN), jnp.float32),
            pltpu.VMEM((CHUNK, HIDDEN), jnp.float32),
            pltpu.SemaphoreType.DMA,
            pltpu.SemaphoreType.DMA,
            pltpu.SemaphoreType.DMA,
            pltpu.SemaphoreType.DMA,
            pltpu.SemaphoreType.DMA,
            pltpu.SemaphoreType.DMA,
        ],
    )
    return f(ids, wtab, ptt, gamma, beta)


def kernel(input_ids, word_embeddings, position_embeddings,
           token_type_embeddings, ln_gamma, ln_beta):
    ids = input_ids.reshape(-1).astype(jnp.int32)
    ptt = _fold_tt(position_embeddings, token_type_embeddings)
    out = _run(ids, word_embeddings, ptt, ln_gamma, ln_beta)
    return out.reshape(input_ids.shape + (HIDDEN,))


# speedup vs baseline: 1.7159x; 1.0357x over previous
import functools

import jax
import jax.numpy as jnp
from jax import lax
from jax.experimental import pallas as pl
from jax.experimental.pallas import tpu as pltpu
from jax.experimental.pallas import tpu_sc as plsc

HIDDEN = 768
PAD = 1
EPS = 1e-5
MAX_POS = 2050

NC, NS, L = 2, 16, 16
NW = NC * NS
B = 4 * 2048
ROWS_PER_W = B // NW
CHUNK = 16
NCHUNK = ROWS_PER_W // CHUNK
NG = HIDDEN // L


def _gs(g):
    return pl.ds(g * L, L)


def _vrsqrt(v):
    i = plsc.bitcast(v, jnp.int32)
    i = jnp.int32(0x5F3759DF) - (i >> 1)
    y = plsc.bitcast(i, jnp.float32)
    for _ in range(3):
        y = y * (1.5 - 0.5 * v * y * y)
    return y


def _fold_body(pos_ref, tt_ref, o_ref):
    o_ref[...] = pos_ref[...] + tt_ref[0:1, :]


@jax.jit
def _fold_tt(ptab, tttab):
    return pl.pallas_call(
        _fold_body,
        out_shape=jax.ShapeDtypeStruct((MAX_POS, HIDDEN), jnp.float32),
    )(ptab, tttab)


def _sc_body(ids_hbm, wtab_hbm, ptt_hbm, gamma_hbm, beta_hbm,
             out_hbm,
             ids_v, wida, pida, gam_v, bet_v, pref_v,
             wbuf0, wbuf1, pbuf0, pbuf1, obuf0, obuf1,
             wsem0, wsem1, psem0, psem1, osem0, osem1):
    c = lax.axis_index("c")
    s = lax.axis_index("s")
    wid = c * NS + s
    chk = wid % 8
    base_tok = wid * ROWS_PER_W
    seq_tok = base_tok - chk * ROWS_PER_W

    pltpu.sync_copy(ids_hbm.at[pl.ds(base_tok, ROWS_PER_W)], ids_v)
    pltpu.sync_copy(gamma_hbm, gam_v)
    pltpu.sync_copy(beta_hbm, bet_v)

    pltpu.sync_copy(ids_hbm.at[pl.ds(seq_tok, 7 * ROWS_PER_W)], pref_v)

    def cbody(i, a):
        v = pref_v[pl.ds(i * L, L)]
        return a + jnp.where(v != PAD, 1, 0).astype(jnp.int32)

    off = lax.fori_loop(0, chk * (ROWS_PER_W // L), cbody,
                        jnp.zeros((L,), jnp.int32))
    off = jnp.full((L,), jnp.sum(off), jnp.int32)

    def pbody(i, run):
        v = ids_v[pl.ds(i * L, L)]
        m = jnp.where(v != PAD, 1, 0).astype(jnp.int32)
        cs = plsc.cumsum(m)
        wida[i] = v
        pida[i] = (cs + run) * m + PAD
        return run + jnp.sum(m)

    lax.fori_loop(0, NCHUNK, pbody, off)

    wbufs = (wbuf0, wbuf1)
    pbufs = (pbuf0, pbuf1)
    obufs = (obuf0, obuf1)
    wsems = (wsem0, wsem1)
    psems = (psem0, psem1)
    osems = (osem0, osem1)
    inv_h = jnp.float32(1.0 / HIDDEN)
    z = jnp.zeros((L,), jnp.float32)

    def chunk_step(k, b):
        wb, pb, ob = wbufs[b], pbufs[b], obufs[b]
        pltpu.make_async_copy(ob, out_hbm.at[pl.ds(base_tok, CHUNK)],
                              osems[b]).wait()
        pltpu.make_async_copy(wtab_hbm.at[wida.at[k]], wb, wsems[b]).wait()
        pltpu.make_async_copy(ptt_hbm.at[pida.at[k]], pb, psems[b]).wait()

        BW = 8

        def rbody(r, _):
            xs = []
            s0 = s1 = s2 = s3 = z
            q0 = q1 = q2 = q3 = z
            for gb in range(0, NG, BW):
                ws = [wb[r, _gs(g)] for g in range(gb, gb + BW)]
                ps = [pb[r, _gs(g)] for g in range(gb, gb + BW)]
                xb4 = [w + p for w, p in zip(ws, ps)]
                xs.extend(xb4)
                sq = [x * x for x in xb4]
                s0 = s0 + (xb4[0] + xb4[4])
                s1 = s1 + (xb4[1] + xb4[5])
                s2 = s2 + (xb4[2] + xb4[6])
                s3 = s3 + (xb4[3] + xb4[7])
                q0 = q0 + (sq[0] + sq[4])
                q1 = q1 + (sq[1] + sq[5])
                q2 = q2 + (sq[2] + sq[6])
                q3 = q3 + (sq[3] + sq[7])
            mean = jnp.full((L,), jnp.sum((s0 + s1) + (s2 + s3)) * inv_h,
                            jnp.float32)
            ex2 = jnp.full((L,), jnp.sum((q0 + q1) + (q2 + q3)) * inv_h,
                           jnp.float32)
            rinv = _vrsqrt(ex2 - mean * mean + EPS)
            mb = mean * rinv
            for gb in range(0, NG, BW):
                gvs = [gam_v[_gs(g)] for g in range(gb, gb + BW)]
                bvs = [bet_v[_gs(g)] for g in range(gb, gb + BW)]
                ts = [xs[g] * rinv - mb for g in range(gb, gb + BW)]
                ys = [t * gv + bv for t, gv, bv in zip(ts, gvs, bvs)]
                for i, g in enumerate(range(gb, gb + BW)):
                    ob[r, _gs(g)] = ys[i]
            return 0

        lax.fori_loop(0, CHUNK, rbody, 0)

        pltpu.async_copy(ob, out_hbm.at[pl.ds(base_tok + k * CHUNK, CHUNK)],
                         osems[b])
        k2 = (k + 2) % NCHUNK
        pltpu.async_copy(wtab_hbm.at[wida.at[k2]], wb, wsems[b])
        pltpu.async_copy(ptt_hbm.at[pida.at[k2]], pb, psems[b])

    pltpu.async_copy(wtab_hbm.at[wida.at[0]], wbuf0, wsem0)
    pltpu.async_copy(ptt_hbm.at[pida.at[0]], pbuf0, psem0)
    pltpu.async_copy(wtab_hbm.at[wida.at[1]], wbuf1, wsem1)
    pltpu.async_copy(ptt_hbm.at[pida.at[1]], pbuf1, psem1)
    pltpu.async_copy(obuf0, out_hbm.at[pl.ds(base_tok, CHUNK)], osem0)
    pltpu.async_copy(obuf1, out_hbm.at[pl.ds(base_tok, CHUNK)], osem1)

    def loop_body(i, _):
        k = 2 * i
        chunk_step(k, 0)
        chunk_step(k + 1, 1)
        return 0

    lax.fori_loop(0, NCHUNK // 2, loop_body, 0)

    pltpu.make_async_copy(wtab_hbm.at[wida.at[0]], wbuf0, wsem0).wait()
    pltpu.make_async_copy(ptt_hbm.at[pida.at[0]], pbuf0, psem0).wait()
    pltpu.make_async_copy(wtab_hbm.at[wida.at[1]], wbuf1, wsem1).wait()
    pltpu.make_async_copy(ptt_hbm.at[pida.at[1]], pbuf1, psem1).wait()
    pltpu.make_async_copy(obuf0, out_hbm.at[pl.ds(base_tok, CHUNK)],
                          osem0).wait()
    pltpu.make_async_copy(obuf1, out_hbm.at[pl.ds(base_tok, CHUNK)],
                          osem1).wait()


@jax.jit
def _run(ids, wtab, ptt, gamma, beta):
    mesh = plsc.VectorSubcoreMesh(
        core_axis_name="c", subcore_axis_name="s",
        num_cores=NC, num_subcores=NS)
    f = pl.kernel(
        _sc_body,
        out_type=jax.ShapeDtypeStruct((B, HIDDEN), jnp.float32),
        mesh=mesh,
        compiler_params=pltpu.CompilerParams(needs_layout_passes=False),
        scratch_types=[
            pltpu.VMEM((ROWS_PER_W,), jnp.int32),
            pltpu.VMEM((NCHUNK, CHUNK), jnp.int32),
            pltpu.VMEM((NCHUNK, CHUNK), jnp.int32),
            pltpu.VMEM((HIDDEN,), jnp.float32),
            pltpu.VMEM((HIDDEN,), jnp.float32),
            pltpu.VMEM((7 * ROWS_PER_W,), jnp.int32),
            pltpu.VMEM((CHUNK, HIDDEN), jnp.float32),
            pltpu.VMEM((CHUNK, HIDDEN), jnp.float32),
            pltpu.VMEM((CHUNK, HIDDEN), jnp.float32),
            pltpu.VMEM((CHUNK, HIDDEN), jnp.float32),
            pltpu.VMEM((CHUNK, HIDDEN), jnp.float32),
            pltpu.VMEM((CHUNK, HIDDEN), jnp.float32),
            pltpu.SemaphoreType.DMA,
            pltpu.SemaphoreType.DMA,
            pltpu.SemaphoreType.DMA,
            pltpu.SemaphoreType.DMA,
            pltpu.SemaphoreType.DMA,
            pltpu.SemaphoreType.DMA,
        ],
    )
    return f(ids, wtab, ptt, gamma, beta)


def kernel(input_ids, word_embeddings, position_embeddings,
           token_type_embeddings, ln_gamma, ln_beta):
    ids = input_ids.reshape(-1).astype(jnp.int32)
    ptt = _fold_tt(position_embeddings, token_type_embeddings)
    out = _run(ids, word_embeddings, ptt, ln_gamma, ln_beta)
    return out.reshape(input_ids.shape + (HIDDEN,))
